# baseline (speedup 1.0000x reference)
import functools

import jax
import jax.numpy as jnp
from jax import lax
from jax.experimental import pallas as pl
from jax.experimental.pallas import tpu as pltpu
from jax.experimental.pallas import tpu_sc as plsc

_BATCH = 4096
_HIST = 200
_EMB = 64
_NW = 32
_BPW = _BATCH // _NW
_C0 = 104
_C1 = _HIST - _C0

_mesh = plsc.VectorSubcoreMesh(core_axis_name="c", subcore_axis_name="s")


@functools.partial(
    pl.kernel,
    mesh=_mesh,
    out_type=jax.ShapeDtypeStruct((_BATCH, _EMB), jnp.float32),
    scratch_types=[
        pltpu.VMEM((_BPW * _HIST,), jnp.int32),
        pltpu.VMEM((_HIST, _EMB), jnp.float32),
        pltpu.VMEM((_BPW, _EMB), jnp.float32),
        pltpu.SemaphoreType.DMA,
    ],
    compiler_params=pltpu.CompilerParams(use_tc_tiling_on_sc=False),
)
def _sum_embed(idx_hbm, table_hbm, out_hbm, idx_v, rows_v, out_v, sem):
    wid = lax.axis_index("s") * 2 + lax.axis_index("c")
    base = wid * _BPW
    pltpu.sync_copy(idx_hbm.at[pl.ds(base * _HIST, _BPW * _HIST)], idx_v)

    def row_body(r, carry):
        off = r * _HIST
        pltpu.async_copy(
            table_hbm.at[idx_v.at[pl.ds(off, _C0)]],
            rows_v.at[pl.ds(0, _C0)],
            sem,
        ).wait()
        pltpu.async_copy(
            table_hbm.at[idx_v.at[pl.ds(off + _C0, _C1)]],
            rows_v.at[pl.ds(_C0, _C1)],
            sem,
        ).wait()

        def acc_body(j, accs):
            return tuple(
                accs[d] + rows_v[j, pl.ds(d * 16, 16)] for d in range(4)
            )

        zero = jnp.zeros((16,), jnp.float32)
        accs = lax.fori_loop(0, _HIST, acc_body, (zero, zero, zero, zero))
        for d in range(4):
            out_v[r, pl.ds(d * 16, 16)] = accs[d]
        return carry

    lax.fori_loop(0, _BPW, row_body, 0)
    pltpu.sync_copy(out_v, out_hbm.at[pl.ds(base, _BPW)])


def _linear_body(s_ref, wt_ref, b_ref, o_ref):
    o_ref[...] = (
        jnp.dot(s_ref[...], wt_ref[...], preferred_element_type=jnp.float32)
        + b_ref[...]
    )


def _linear(sums, Wt, b2d):
    return pl.pallas_call(
        _linear_body,
        out_shape=jax.ShapeDtypeStruct((_BATCH, Wt.shape[1]), jnp.float32),
    )(sums, Wt, b2d)


@jax.jit
def kernel(input, emb_table, W, b):
    idx = input.reshape(-1)
    sums = _sum_embed(idx, emb_table)
    out = _linear(sums, W.T, b.reshape(1, -1))
    return out


# speedup vs baseline: 1.3000x; 1.3000x over previous
import functools

import jax
import jax.numpy as jnp
from jax import lax
from jax.experimental import pallas as pl
from jax.experimental.pallas import tpu as pltpu
from jax.experimental.pallas import tpu_sc as plsc

_BATCH = 4096
_HIST = 200
_EMB = 64
_NW = 32
_BPW = _BATCH // _NW
_C0 = 104
_C1 = _HIST - _C0

_NBUF = 4
_GRP = _BPW // _NBUF

_mesh = plsc.VectorSubcoreMesh(core_axis_name="c", subcore_axis_name="s")


@functools.partial(
    pl.kernel,
    mesh=_mesh,
    out_type=jax.ShapeDtypeStruct((_BATCH, _EMB), jnp.float32),
    scratch_types=[
        pltpu.VMEM((_BPW * _HIST,), jnp.int32),
        pltpu.VMEM((_NBUF, _HIST, _EMB), jnp.float32),
        pltpu.VMEM((_BPW, _EMB), jnp.float32),
        pltpu.SemaphoreType.DMA((_NBUF,)),
    ],
    compiler_params=pltpu.CompilerParams(use_tc_tiling_on_sc=False),
)
def _sum_embed(idx_hbm, table_hbm, out_hbm, idx_v, bufs, out_v, sems):
    wid = lax.axis_index("s") * 2 + lax.axis_index("c")
    base = wid * _BPW
    pltpu.sync_copy(idx_hbm.at[pl.ds(base * _HIST, _BPW * _HIST)], idx_v)

    def fire(r, slot):
        off = r * _HIST
        pltpu.async_copy(
            table_hbm.at[idx_v.at[pl.ds(off, _C0)]],
            bufs.at[slot, pl.ds(0, _C0)],
            sems.at[slot],
        )
        pltpu.async_copy(
            table_hbm.at[idx_v.at[pl.ds(off + _C0, _C1)]],
            bufs.at[slot, pl.ds(_C0, _C1)],
            sems.at[slot],
        )

    def drain(r, slot):
        off = r * _HIST
        pltpu.make_async_copy(
            table_hbm.at[idx_v.at[pl.ds(off, _C0)]],
            bufs.at[slot, pl.ds(0, _C0)],
            sems.at[slot],
        ).wait()
        pltpu.make_async_copy(
            table_hbm.at[idx_v.at[pl.ds(off + _C0, _C1)]],
            bufs.at[slot, pl.ds(_C0, _C1)],
            sems.at[slot],
        ).wait()

    for s in range(_NBUF):
        fire(s, s)

    def group_body(g, carry):
        for s in range(_NBUF):
            r = g * _NBUF + s
            drain(r, s)

            @pl.when(r + _NBUF < _BPW)
            def _():
                fire(r + _NBUF, s)

            def acc_body(j, accs):
                return tuple(
                    accs[d] + bufs[s, j, pl.ds(d * 16, 16)] for d in range(4)
                )

            zero = jnp.zeros((16,), jnp.float32)
            accs = lax.fori_loop(0, _HIST, acc_body, (zero, zero, zero, zero))
            for d in range(4):
                out_v[r, pl.ds(d * 16, 16)] = accs[d]
        return carry

    lax.fori_loop(0, _GRP, group_body, 0)
    pltpu.sync_copy(out_v, out_hbm.at[pl.ds(base, _BPW)])


def _linear_body(s_ref, wt_ref, b_ref, o_ref):
    o_ref[...] = (
        jnp.dot(s_ref[...], wt_ref[...], preferred_element_type=jnp.float32)
        + b_ref[...]
    )


def _linear(sums, Wt, b2d):
    return pl.pallas_call(
        _linear_body,
        out_shape=jax.ShapeDtypeStruct((_BATCH, Wt.shape[1]), jnp.float32),
    )(sums, Wt, b2d)


@jax.jit
def kernel(input, emb_table, W, b):
    idx = input.reshape(-1)
    sums = _sum_embed(idx, emb_table)
    out = _linear(sums, W.T, b.reshape(1, -1))
    return out


# speedup vs baseline: 1.3382x; 1.0294x over previous
import functools

import jax
import jax.numpy as jnp
from jax import lax
from jax.experimental import pallas as pl
from jax.experimental.pallas import tpu as pltpu
from jax.experimental.pallas import tpu_sc as plsc

_BATCH = 4096
_HIST = 200
_EMB = 64
_NW = 32
_BPW = _BATCH // _NW
_C0 = 104
_C1 = _HIST - _C0

_NBUF = 4
_GRP = _BPW // _NBUF

_mesh = plsc.VectorSubcoreMesh(core_axis_name="c", subcore_axis_name="s")


@functools.partial(
    pl.kernel,
    mesh=_mesh,
    out_type=jax.ShapeDtypeStruct((_BATCH, _EMB), jnp.float32),
    scratch_types=[
        pltpu.VMEM((_BPW * _HIST,), jnp.int32),
        pltpu.VMEM((_NBUF, _HIST, _EMB), jnp.float32),
        pltpu.VMEM((_BPW, _EMB), jnp.float32),
        pltpu.SemaphoreType.DMA((_NBUF,)),
    ],
    compiler_params=pltpu.CompilerParams(use_tc_tiling_on_sc=False),
)
def _sum_embed(idx_hbm, table_hbm, out_hbm, idx_v, bufs, out_v, sems):
    wid = lax.axis_index("s") * 2 + lax.axis_index("c")
    base = wid * _BPW
    pltpu.sync_copy(idx_hbm.at[pl.ds(base * _HIST, _BPW * _HIST)], idx_v)

    def fire(r, slot):
        off = r * _HIST
        pltpu.async_copy(
            table_hbm.at[idx_v.at[pl.ds(off, _C0)]],
            bufs.at[slot, pl.ds(0, _C0)],
            sems.at[slot],
        )
        pltpu.async_copy(
            table_hbm.at[idx_v.at[pl.ds(off + _C0, _C1)]],
            bufs.at[slot, pl.ds(_C0, _C1)],
            sems.at[slot],
        )

    def drain(r, slot):
        off = r * _HIST
        pltpu.make_async_copy(
            table_hbm.at[idx_v.at[pl.ds(off, _C0)]],
            bufs.at[slot, pl.ds(0, _C0)],
            sems.at[slot],
        ).wait()
        pltpu.make_async_copy(
            table_hbm.at[idx_v.at[pl.ds(off + _C0, _C1)]],
            bufs.at[slot, pl.ds(_C0, _C1)],
            sems.at[slot],
        ).wait()

    for s in range(_NBUF):
        fire(s, s)

    def group_body(g, carry):
        for s in range(_NBUF):
            r = g * _NBUF + s
            drain(r, s)

            @pl.when(r + _NBUF < _BPW)
            def _():
                fire(r + _NBUF, s)

            def acc_body(j8, accs):
                a = list(accs)
                jb = j8 * 8
                for u in range(8):
                    for d in range(4):
                        a[d] = a[d] + bufs[s, jb + u, pl.ds(d * 16, 16)]
                return tuple(a)

            zero = jnp.zeros((16,), jnp.float32)
            accs = lax.fori_loop(0, _HIST // 8, acc_body, (zero,) * 4)
            for d in range(4):
                out_v[r, pl.ds(d * 16, 16)] = accs[d]
        return carry

    lax.fori_loop(0, _GRP, group_body, 0)
    pltpu.sync_copy(out_v, out_hbm.at[pl.ds(base, _BPW)])


def _linear_body(s_ref, wt_ref, b_ref, o_ref):
    o_ref[...] = (
        jnp.dot(s_ref[...], wt_ref[...], preferred_element_type=jnp.float32)
        + b_ref[...]
    )


def _linear(sums, Wt, b2d):
    return pl.pallas_call(
        _linear_body,
        out_shape=jax.ShapeDtypeStruct((_BATCH, Wt.shape[1]), jnp.float32),
    )(sums, Wt, b2d)


@jax.jit
def kernel(input, emb_table, W, b):
    idx = input.reshape(-1)
    sums = _sum_embed(idx, emb_table)
    out = _linear(sums, W.T, b.reshape(1, -1))
    return out
